# baseline (speedup 1.0000x reference)
import jax
import jax.numpy as jnp
import numpy as np
from jax.experimental import pallas as pl
from jax.experimental.pallas import tpu as pltpu

E = 64
H = 64
NH = 8
HD = H // NH
T2 = 2048
SCALE = 1.0 / np.sqrt(HD)


def _gelu(x):
    return 0.5 * x * (1.0 + jax.lax.erf(x * np.float32(1.0 / np.sqrt(2.0))))


def _mmT(x, w):
    return jax.lax.dot_general(x, w, (((1,), (1,)), ((), ())),
                               preferred_element_type=jnp.float32)


def _router_kernel(cfa_ref, cfb_ref, hfa_ref, hfb_ref,
                   wc_ref, bc_ref, gc_ref, bec_ref,
                   wh_ref, bh_ref, gh_ref, beh_ref,
                   wi_ref, bi_ref, wo_ref, bo_ref,
                   wf_ref, bf_ref, gf_ref, bef_ref,
                   w1_ref, b1_ref, w2_ref, b2_ref,
                   wu1_ref, bu1_ref, wu2_ref, bu2_ref,
                   m1_ref, sels_ref, pf0_ref, pf1_ref, pf2_ref, pf3_ref,
                   pb1_ref, pb2_ref, pb3_ref, pb4_ref,
                   q48_ref, r48_ref, mc_ref, mh_ref, mt_ref,
                   rb_ref, unc_ref,
                   sc_qkv, sc_o, sc_f, sc_head, sc_rb, sc_un,
                   sc_b1, sc_b2, sc_b3, sc_b4, sc_stt,
                   sc_wc, sc_wh, sc_bc, sc_bh, sc_bf):
    f32 = jnp.float32
    mm = lambda a, b: jnp.dot(a, b, preferred_element_type=f32)
    dup = lambda v: jnp.concatenate([v, v], axis=1)

    def _assemble():
        ones64 = jnp.full((1, H), 1.0 / H, f32)
        col64 = jnp.full((H, 1), 1.0 / H, f32)

        wcv = wc_ref[...]
        sc_wc[...] = wcv - mm(ones64, wcv)
        whv = wh_ref[...]
        sc_wh[...] = whv - mm(ones64, whv)
        bcr = bc_ref[...].reshape(1, -1)
        sc_bc[...] = dup(bcr - mm(bcr, col64))
        bhr = bh_ref[...].reshape(1, -1)
        sc_bh[...] = dup(bhr - mm(bhr, col64))
        bfr = bf_ref[...].reshape(1, -1)
        sc_bf[...] = dup(bfr - mm(bfr, col64))

        wi = wi_ref[...]
        sc_qkv[...] = jnp.zeros((3 * 2 * H, 2 * H), f32)
        for j in range(3):
            blk = wi[j * H:(j + 1) * H, :]
            sc_qkv[2 * j * H:(2 * j + 1) * H, 0:H] = blk
            sc_qkv[(2 * j + 1) * H:(2 * j + 2) * H, H:2 * H] = blk
        sc_o[...] = jnp.zeros((2 * H, 2 * H), f32)
        sc_o[0:H, 0:H] = wo_ref[...]
        sc_o[H:2 * H, H:2 * H] = wo_ref[...]
        wfv = wf_ref[...]
        wfc = wfv - mm(ones64, wfv)
        sc_f[...] = jnp.zeros((2 * H, 2 * H), f32)
        sc_f[0:H, 0:H] = wfc
        sc_f[H:2 * H, H:2 * H] = wfc
        sc_head[...] = jnp.zeros((96, 2 * H), f32)
        sc_head[0:32, 0:H] = w1_ref[...]
        sc_head[32:64, H:2 * H] = w1_ref[...]
        sc_head[64:80, 0:H] = wu1_ref[...]
        sc_head[80:96, H:2 * H] = wu1_ref[...]
        sc_rb[...] = jnp.zeros((2 * H, H), f32)
        sc_rb[0:H, 0:32] = w2_ref[...]
        sc_rb[H:2 * H, 32:64] = w2_ref[...]
        sc_un[...] = jnp.zeros((2 * H, 32), f32)
        sc_un[0:H, 0:16] = wu2_ref[...]
        sc_un[H:2 * H, 16:32] = wu2_ref[...]

        bi2 = bi_ref[...].reshape(1, -1)
        bqr = dup(bi2[:, 0:H])
        bkr = dup(bi2[:, H:2 * H])
        ii = jax.lax.broadcasted_iota(jnp.int32, (2 * H, 2 * H), 0)
        jj = jax.lax.broadcasted_iota(jnp.int32, (2 * H, 2 * H), 1)
        ident = (ii == jj).astype(f32)
        diag_bk = ident * bkr
        diag_bq = ident * bqr
        sc_b1[...] = mm(diag_bk, pb1_ref[...])
        sc_b2[...] = mm(diag_bk, pb2_ref[...])
        sc_b3[...] = mm(diag_bq, pb3_ref[...])
        sc_b4[...] = mm(diag_bq, pb4_ref[...])
        stt16 = mm(bqr * bkr, sels_ref[...])
        sc_stt[...] = jnp.zeros((1, 2 * H), f32)
        sc_stt[0:1, 96:112] = stt16
        sc_stt[0:1, 112:128] = stt16

    _assemble()

    m1 = m1_ref[...]

    def segln_c(c, g, b):
        v = mm(c * c, m1)
        return c * jax.lax.rsqrt(v + 1e-5) * dup(g.reshape(1, -1)) \
            + dup(b.reshape(1, -1))

    wc = sc_wc[...]
    pre_c = jnp.concatenate([_mmT(cfa_ref[...], wc), _mmT(cfb_ref[...], wc)],
                            axis=1) + sc_bc[...]
    ce = _gelu(segln_c(pre_c, gc_ref[...], bec_ref[...]))
    wh = sc_wh[...]
    pre_h = jnp.concatenate([_mmT(hfa_ref[...], wh), _mmT(hfb_ref[...], wh)],
                            axis=1) + sc_bh[...]
    he = _gelu(segln_c(pre_h, gh_ref[...], beh_ref[...]))

    bi2 = bi_ref[...].reshape(1, -1)
    bq = dup(bi2[:, 0:H]); bk = dup(bi2[:, H:2 * H]); bv = dup(bi2[:, 2 * H:])
    bqkv = jnp.concatenate([bq, bk, bv], axis=1)
    qkv_c = _mmT(ce, sc_qkv[...]) + bqkv
    qkv_h = _mmT(he, sc_qkv[...]) + bqkv
    q_c = qkv_c[:, 0:128]; k_c = qkv_c[:, 128:256]; v_c = qkv_c[:, 256:384]
    q_h = qkv_h[:, 0:128]; k_h = qkv_h[:, 128:256]; v_h = qkv_h[:, 256:384]

    d = (mm(q_c * k_c, pf0_ref[...]) + mm(q_c * k_h, pf1_ref[...])
         + mm(q_h * k_c, pf2_ref[...]) + mm(q_h * k_h, pf3_ref[...])
         + mm(q_c, sc_b1[...]) + mm(q_h, sc_b2[...])
         + mm(k_c, sc_b3[...]) + mm(k_h, sc_b4[...])) - sc_stt[...]
    e = jnp.exp(d)
    den = mm(e, q48_ref[...]) + 1.0
    inv3 = 1.0 / den
    a = e * mm(inv3, r48_ref[...])
    o = (mm(a, mc_ref[...]) * v_c + mm(a, mh_ref[...]) * v_h
         + mm(inv3, mt_ref[...]) * bv)
    att_mean = _mmT(o, sc_o[...]) + dup(bo_ref[...].reshape(1, -1))

    fused = _gelu(segln_c(_mmT(att_mean, sc_f[...]) + sc_bf[...],
                          gf_ref[...], bef_ref[...]))

    bhd = jnp.concatenate([dup(b1_ref[...].reshape(1, -1)),
                           dup(bu1_ref[...].reshape(1, -1))], axis=1)
    hh = _gelu(_mmT(fused, sc_head[...]) + bhd)
    h1 = hh[:, 0:64]
    hu = hh[:, 64:96]
    rb = jnp.tanh(_mmT(h1, sc_rb[...])
                  + dup(b2_ref[...].reshape(1, -1)))
    unc = jnp.logaddexp(_mmT(hu, sc_un[...])
                        + dup(bu2_ref[...].reshape(1, -1)), 0.0)

    rb_ref[0:T2, :] = rb[:, 0:E]
    rb_ref[T2:2 * T2, :] = rb[:, E:2 * E]
    unc_ref[0:T2, :] = unc[:, 0:E]
    unc_ref[T2:2 * T2, :] = unc[:, E:2 * E]


def _consts():
    f32 = np.float32
    hl = np.arange(128) // HD
    sels = np.zeros((128, 16), f32)
    sels[np.arange(128), hl] = SCALE

    def place(o):
        m = np.zeros((128, 128), f32)
        m[np.arange(128), o * 16 + hl] = SCALE
        return m

    psh = np.zeros((128, 128), f32)
    j = np.arange(128)
    for lo, hi, base in ((0, 16, 64), (16, 32, 64), (32, 48, 80), (48, 64, 80)):
        cols = j[(j >= lo) & (j < hi)]
        psh[base + (cols % 16), cols] = 1.0
    psh[j[(j >= 64) & (j < 96)], j[(j >= 64) & (j < 96)]] = 1.0
    ish = np.eye(128, dtype=f32) - psh

    q48 = np.zeros((128, 48), f32)
    for jq in range(16):
        q48[jq, jq] = 1.0; q48[16 + jq, jq] = 1.0
        q48[32 + jq, 16 + jq] = 1.0; q48[48 + jq, 16 + jq] = 1.0
        q48[96 + jq, 32 + jq] = 1.0; q48[112 + jq, 32 + jq] = 1.0

    r48 = np.zeros((48, 128), f32)
    r48[j[:32] % 16, j[:32]] = 1.0
    r48[16 + (j[32:64] % 16), j[32:64]] = 1.0
    r48[32 + (j[96:128] % 16), j[96:128]] = 1.0

    third = np.float32(1.0 / 3.0)
    mc = np.zeros((128, 128), f32)
    mh = np.zeros((128, 128), f32)
    mt = np.zeros((48, 128), f32)
    for lane in range(128):
        h = lane // HD
        for r in (h, 32 + h, 96 + h):
            mc[r, lane] = third
        for r in (16 + h, 48 + h, 112 + h):
            mh[r, lane] = third
        for r in (h, 16 + h, 32 + h):
            mt[r, lane] = third

    m1 = np.zeros((128, 128), f32)
    m1[:H, :H] = 1.0 / H
    m1[H:, H:] = 1.0 / H
    return [jnp.asarray(x) for x in
            (m1, sels,
             place(0) @ ish, place(1) @ ish, place(2) @ ish, place(3) @ ish,
             place(4) @ ish, place(5) @ ish, place(6) @ ish, place(7) @ ish,
             q48, r48, mc, mh, mt)]


@jax.jit
def kernel(cost_features, hardware_features, w_cost, b_cost, g_cost, be_cost,
           w_hw, b_hw, g_hw, be_hw, in_proj_w, in_proj_b, out_proj_w,
           out_proj_b, w_fus, b_fus, g_fus, be_fus, w_out1, b_out1, w_out2,
           b_out2, w_unc1, b_unc1, w_unc2, b_unc2):
    B, CD = cost_features.shape
    grid = (B // (2 * T2),)

    operands = [
        cost_features, cost_features, hardware_features, hardware_features,
        w_cost, b_cost, g_cost, be_cost,
        w_hw, b_hw, g_hw, be_hw,
        in_proj_w, in_proj_b, out_proj_w, out_proj_b,
        w_fus, b_fus, g_fus, be_fus,
        w_out1, b_out1, w_out2, b_out2,
        w_unc1, b_unc1, w_unc2, b_unc2,
    ] + _consts()
    full = lambda a: pl.BlockSpec(a.shape, lambda i: (0,) * a.ndim)
    in_specs = [pl.BlockSpec((T2, CD), lambda i: (2 * i, 0)),
                pl.BlockSpec((T2, CD), lambda i: (2 * i + 1, 0)),
                pl.BlockSpec((T2, 8), lambda i: (2 * i, 0)),
                pl.BlockSpec((T2, 8), lambda i: (2 * i + 1, 0))]
    in_specs += [full(a) for a in operands[4:]]

    out_shape = [jax.ShapeDtypeStruct((B, E), jnp.float32),
                 jax.ShapeDtypeStruct((B, E), jnp.float32)]
    out_specs = [pl.BlockSpec((2 * T2, E), lambda i: (i, 0)),
                 pl.BlockSpec((2 * T2, E), lambda i: (i, 0))]

    scratch_shapes = [
        pltpu.VMEM((3 * 2 * H, 2 * H), jnp.float32),
        pltpu.VMEM((2 * H, 2 * H), jnp.float32),
        pltpu.VMEM((2 * H, 2 * H), jnp.float32),
        pltpu.VMEM((96, 2 * H), jnp.float32),
        pltpu.VMEM((2 * H, H), jnp.float32),
        pltpu.VMEM((2 * H, 32), jnp.float32),
        pltpu.VMEM((2 * H, 2 * H), jnp.float32),
        pltpu.VMEM((2 * H, 2 * H), jnp.float32),
        pltpu.VMEM((2 * H, 2 * H), jnp.float32),
        pltpu.VMEM((2 * H, 2 * H), jnp.float32),
        pltpu.VMEM((1, 2 * H), jnp.float32),
        pltpu.VMEM((H, 6 * H), jnp.float32),
        pltpu.VMEM((H, 8), jnp.float32),
        pltpu.VMEM((1, 2 * H), jnp.float32),
        pltpu.VMEM((1, 2 * H), jnp.float32),
        pltpu.VMEM((1, 2 * H), jnp.float32),
    ]

    rb, unc = pl.pallas_call(
        _router_kernel,
        grid=grid,
        in_specs=in_specs,
        out_specs=out_specs,
        out_shape=out_shape,
        scratch_shapes=scratch_shapes,
        compiler_params=pltpu.CompilerParams(
            dimension_semantics=("parallel",)),
    )(*operands)
    return rb, unc


# speedup vs baseline: 1.0276x; 1.0276x over previous
import jax
import jax.numpy as jnp
import numpy as np
from jax.experimental import pallas as pl
from jax.experimental.pallas import tpu as pltpu

E = 64
H = 64
NH = 8
HD = H // NH
T2 = 2048
SCALE = 1.0 / np.sqrt(HD)


def _gelu(x):
    return 0.5 * x * (1.0 + jax.lax.erf(x * np.float32(1.0 / np.sqrt(2.0))))


def _mmT(x, w):
    return jax.lax.dot_general(x, w, (((1,), (1,)), ((), ())),
                               preferred_element_type=jnp.float32)


def _router_kernel(cfa_ref, cfb_ref, hfa_ref, hfb_ref,
                   wc_ref, bc_ref, gc_ref, bec_ref,
                   wh_ref, bh_ref, gh_ref, beh_ref,
                   wi_ref, bi_ref, wo_ref, bo_ref,
                   wf_ref, bf_ref, gf_ref, bef_ref,
                   w1_ref, b1_ref, w2_ref, b2_ref,
                   wu1_ref, bu1_ref, wu2_ref, bu2_ref,
                   m1_ref, sels_ref, pf0_ref, pf1_ref, pf2_ref, pf3_ref,
                   pb1_ref, pb2_ref, pb3_ref, pb4_ref,
                   q48_ref, r48_ref, mc_ref, mh_ref, mt_ref,
                   rb_ref, unc_ref,
                   sc_qkv, sc_f, sc_head, sc_rb, sc_un,
                   sc_b1, sc_b2, sc_b3, sc_b4, sc_stt,
                   sc_wc, sc_wh, sc_bc, sc_bh, sc_bf):
    f32 = jnp.float32
    mm = lambda a, b: jnp.dot(a, b, preferred_element_type=f32)
    dup = lambda v: jnp.concatenate([v, v], axis=1)

    @pl.when(pl.program_id(0) == 0)
    def _assemble():
        ones64 = jnp.full((1, H), 1.0 / H, f32)
        col64 = jnp.full((H, 1), 1.0 / H, f32)

        wcv = wc_ref[...]
        sc_wc[...] = wcv - mm(ones64, wcv)
        whv = wh_ref[...]
        sc_wh[...] = whv - mm(ones64, whv)
        bcr = bc_ref[...].reshape(1, -1)
        sc_bc[...] = dup(bcr - mm(bcr, col64))
        bhr = bh_ref[...].reshape(1, -1)
        sc_bh[...] = dup(bhr - mm(bhr, col64))
        bfr = bf_ref[...].reshape(1, -1)
        bor = bo_ref[...].reshape(1, -1)

        wi = wi_ref[...]
        sc_qkv[...] = jnp.zeros((3 * 2 * H, 2 * H), f32)
        for j in range(3):
            blk = wi[j * H:(j + 1) * H, :]
            sc_qkv[2 * j * H:(2 * j + 1) * H, 0:H] = blk
            sc_qkv[(2 * j + 1) * H:(2 * j + 2) * H, H:2 * H] = blk
        wfv = wf_ref[...]
        wfc = wfv - mm(ones64, wfv)
        wfo = mm(wfc, wo_ref[...])
        sc_f[...] = jnp.zeros((2 * H, 2 * H), f32)
        sc_f[0:H, 0:H] = wfo
        sc_f[H:2 * H, H:2 * H] = wfo
        sc_bf[...] = dup(bfr - mm(bfr, col64) + _mmT(bor, wfc))
        sc_head[...] = jnp.zeros((96, 2 * H), f32)
        sc_head[0:32, 0:H] = w1_ref[...]
        sc_head[32:64, H:2 * H] = w1_ref[...]
        sc_head[64:80, 0:H] = wu1_ref[...]
        sc_head[80:96, H:2 * H] = wu1_ref[...]
        sc_rb[...] = jnp.zeros((2 * H, H), f32)
        sc_rb[0:H, 0:32] = w2_ref[...]
        sc_rb[H:2 * H, 32:64] = w2_ref[...]
        sc_un[...] = jnp.zeros((2 * H, 32), f32)
        sc_un[0:H, 0:16] = wu2_ref[...]
        sc_un[H:2 * H, 16:32] = wu2_ref[...]

        bi2 = bi_ref[...].reshape(1, -1)
        bqr = dup(bi2[:, 0:H])
        bkr = dup(bi2[:, H:2 * H])
        ii = jax.lax.broadcasted_iota(jnp.int32, (2 * H, 2 * H), 0)
        jj = jax.lax.broadcasted_iota(jnp.int32, (2 * H, 2 * H), 1)
        ident = (ii == jj).astype(f32)
        diag_bk = ident * bkr
        diag_bq = ident * bqr
        sc_b1[...] = mm(diag_bk, pb1_ref[...])
        sc_b2[...] = mm(diag_bk, pb2_ref[...])
        sc_b3[...] = mm(diag_bq, pb3_ref[...])
        sc_b4[...] = mm(diag_bq, pb4_ref[...])
        stt16 = mm(bqr * bkr, sels_ref[...])
        sc_stt[...] = jnp.zeros((1, 2 * H), f32)
        sc_stt[0:1, 96:112] = stt16
        sc_stt[0:1, 112:128] = stt16

    m1 = m1_ref[...]

    def segln_c(c, g, b):
        v = mm(c * c, m1)
        return c * jax.lax.rsqrt(v + 1e-5) * dup(g.reshape(1, -1)) \
            + dup(b.reshape(1, -1))

    wc = sc_wc[...]
    pre_c = jnp.concatenate([_mmT(cfa_ref[...], wc), _mmT(cfb_ref[...], wc)],
                            axis=1) + sc_bc[...]
    ce = _gelu(segln_c(pre_c, gc_ref[...], bec_ref[...]))
    wh = sc_wh[...]
    pre_h = jnp.concatenate([_mmT(hfa_ref[...], wh), _mmT(hfb_ref[...], wh)],
                            axis=1) + sc_bh[...]
    he = _gelu(segln_c(pre_h, gh_ref[...], beh_ref[...]))

    bi2 = bi_ref[...].reshape(1, -1)
    bq = dup(bi2[:, 0:H]); bk = dup(bi2[:, H:2 * H]); bv = dup(bi2[:, 2 * H:])
    bqkv = jnp.concatenate([bq, bk, bv], axis=1)
    qkv_c = _mmT(ce, sc_qkv[...]) + bqkv
    qkv_h = _mmT(he, sc_qkv[...]) + bqkv
    q_c = qkv_c[:, 0:128]; k_c = qkv_c[:, 128:256]; v_c = qkv_c[:, 256:384]
    q_h = qkv_h[:, 0:128]; k_h = qkv_h[:, 128:256]; v_h = qkv_h[:, 256:384]

    d = (mm(q_c * k_c, pf0_ref[...]) + mm(q_c * k_h, pf1_ref[...])
         + mm(q_h * k_c, pf2_ref[...]) + mm(q_h * k_h, pf3_ref[...])
         + mm(q_c, sc_b1[...]) + mm(q_h, sc_b2[...])
         + mm(k_c, sc_b3[...]) + mm(k_h, sc_b4[...])) - sc_stt[...]
    e = jnp.exp(d)
    den = mm(e, q48_ref[...]) + 1.0
    inv3 = 1.0 / den
    a = e * mm(inv3, r48_ref[...])
    o = (mm(a, mc_ref[...]) * v_c + mm(a, mh_ref[...]) * v_h
         + mm(inv3, mt_ref[...]) * bv)

    fused = _gelu(segln_c(_mmT(o, sc_f[...]) + sc_bf[...],
                          gf_ref[...], bef_ref[...]))

    bhd = jnp.concatenate([dup(b1_ref[...].reshape(1, -1)),
                           dup(bu1_ref[...].reshape(1, -1))], axis=1)
    hh = _gelu(_mmT(fused, sc_head[...]) + bhd)
    h1 = hh[:, 0:64]
    hu = hh[:, 64:96]
    rb = jnp.tanh(_mmT(h1, sc_rb[...])
                  + dup(b2_ref[...].reshape(1, -1)))
    unc = jnp.logaddexp(_mmT(hu, sc_un[...])
                        + dup(bu2_ref[...].reshape(1, -1)), 0.0)

    rb_ref[0:T2, :] = rb[:, 0:E]
    rb_ref[T2:2 * T2, :] = rb[:, E:2 * E]
    unc_ref[0:T2, :] = unc[:, 0:E]
    unc_ref[T2:2 * T2, :] = unc[:, E:2 * E]


def _consts():
    f32 = np.float32
    hl = np.arange(128) // HD
    sels = np.zeros((128, 16), f32)
    sels[np.arange(128), hl] = SCALE

    def place(o):
        m = np.zeros((128, 128), f32)
        m[np.arange(128), o * 16 + hl] = SCALE
        return m

    psh = np.zeros((128, 128), f32)
    j = np.arange(128)
    for lo, hi, base in ((0, 16, 64), (16, 32, 64), (32, 48, 80), (48, 64, 80)):
        cols = j[(j >= lo) & (j < hi)]
        psh[base + (cols % 16), cols] = 1.0
    psh[j[(j >= 64) & (j < 96)], j[(j >= 64) & (j < 96)]] = 1.0
    ish = np.eye(128, dtype=f32) - psh

    q48 = np.zeros((128, 48), f32)
    for jq in range(16):
        q48[jq, jq] = 1.0; q48[16 + jq, jq] = 1.0
        q48[32 + jq, 16 + jq] = 1.0; q48[48 + jq, 16 + jq] = 1.0
        q48[96 + jq, 32 + jq] = 1.0; q48[112 + jq, 32 + jq] = 1.0

    r48 = np.zeros((48, 128), f32)
    r48[j[:32] % 16, j[:32]] = 1.0
    r48[16 + (j[32:64] % 16), j[32:64]] = 1.0
    r48[32 + (j[96:128] % 16), j[96:128]] = 1.0

    third = np.float32(1.0 / 3.0)
    mc = np.zeros((128, 128), f32)
    mh = np.zeros((128, 128), f32)
    mt = np.zeros((48, 128), f32)
    for lane in range(128):
        h = lane // HD
        for r in (h, 32 + h, 96 + h):
            mc[r, lane] = third
        for r in (16 + h, 48 + h, 112 + h):
            mh[r, lane] = third
        for r in (h, 16 + h, 32 + h):
            mt[r, lane] = third

    m1 = np.zeros((128, 128), f32)
    m1[:H, :H] = 1.0 / H
    m1[H:, H:] = 1.0 / H
    return [jnp.asarray(x) for x in
            (m1, sels,
             place(0) @ ish, place(1) @ ish, place(2) @ ish, place(3) @ ish,
             place(4) @ ish, place(5) @ ish, place(6) @ ish, place(7) @ ish,
             q48, r48, mc, mh, mt)]


@jax.jit
def kernel(cost_features, hardware_features, w_cost, b_cost, g_cost, be_cost,
           w_hw, b_hw, g_hw, be_hw, in_proj_w, in_proj_b, out_proj_w,
           out_proj_b, w_fus, b_fus, g_fus, be_fus, w_out1, b_out1, w_out2,
           b_out2, w_unc1, b_unc1, w_unc2, b_unc2):
    B, CD = cost_features.shape
    grid = (B // (2 * T2),)

    operands = [
        cost_features, cost_features, hardware_features, hardware_features,
        w_cost, b_cost, g_cost, be_cost,
        w_hw, b_hw, g_hw, be_hw,
        in_proj_w, in_proj_b, out_proj_w, out_proj_b,
        w_fus, b_fus, g_fus, be_fus,
        w_out1, b_out1, w_out2, b_out2,
        w_unc1, b_unc1, w_unc2, b_unc2,
    ] + _consts()
    full = lambda a: pl.BlockSpec(a.shape, lambda i: (0,) * a.ndim)
    in_specs = [pl.BlockSpec((T2, CD), lambda i: (2 * i, 0)),
                pl.BlockSpec((T2, CD), lambda i: (2 * i + 1, 0)),
                pl.BlockSpec((T2, 8), lambda i: (2 * i, 0)),
                pl.BlockSpec((T2, 8), lambda i: (2 * i + 1, 0))]
    in_specs += [full(a) for a in operands[4:]]

    out_shape = [jax.ShapeDtypeStruct((B, E), jnp.float32),
                 jax.ShapeDtypeStruct((B, E), jnp.float32)]
    out_specs = [pl.BlockSpec((2 * T2, E), lambda i: (i, 0)),
                 pl.BlockSpec((2 * T2, E), lambda i: (i, 0))]

    scratch_shapes = [
        pltpu.VMEM((3 * 2 * H, 2 * H), jnp.float32),
        pltpu.VMEM((2 * H, 2 * H), jnp.float32),
        pltpu.VMEM((96, 2 * H), jnp.float32),
        pltpu.VMEM((2 * H, H), jnp.float32),
        pltpu.VMEM((2 * H, 32), jnp.float32),
        pltpu.VMEM((2 * H, 2 * H), jnp.float32),
        pltpu.VMEM((2 * H, 2 * H), jnp.float32),
        pltpu.VMEM((2 * H, 2 * H), jnp.float32),
        pltpu.VMEM((2 * H, 2 * H), jnp.float32),
        pltpu.VMEM((1, 2 * H), jnp.float32),
        pltpu.VMEM((H, 6 * H), jnp.float32),
        pltpu.VMEM((H, 8), jnp.float32),
        pltpu.VMEM((1, 2 * H), jnp.float32),
        pltpu.VMEM((1, 2 * H), jnp.float32),
        pltpu.VMEM((1, 2 * H), jnp.float32),
    ]

    rb, unc = pl.pallas_call(
        _router_kernel,
        grid=grid,
        in_specs=in_specs,
        out_specs=out_specs,
        out_shape=out_shape,
        scratch_shapes=scratch_shapes,
    )(*operands)
    return rb, unc
